# Initial kernel scaffold; baseline (speedup 1.0000x reference)
import functools
import math

import jax
import jax.numpy as jnp
from jax import lax
from jax.experimental import pallas as pl
from jax.experimental.pallas import tpu as pltpu
from jax.experimental.pallas import tpu_sc as plsc

DIM = 128
HID = 256
B = 4
N = 65536
T = 10
FH = 4
NHASH = T * FH
ND = 8
NS = 4
NSAMP = ND * NS
K = 5

BN = 1024
NBLK = N // BN
BIG = 1.0e9
ENC = 131072.0


def _topk_body(pts, projT, projM, refc, dvte, scr, offr, out, wm_s, run_s):
    b = pl.program_id(0)
    j = pl.program_id(1)

    @pl.when(jnp.logical_and(b == 0, j == 0))
    def _():
        sT = refc[...] + (dvte[...] * scr[...]) * offr[...]
        ps = jnp.dot(projM[...], sT, preferred_element_type=jnp.float32)
        wm_s[...] = 1.0 - 2.0 * (ps > 0).astype(jnp.float32)

    @pl.when(j == 0)
    def _():
        run_s[...] = jnp.full((8, NSAMP), BIG, jnp.float32)

    x = pts[0]
    proj = jnp.dot(x, projT[...], preferred_element_type=jnp.float32)
    ph = (proj > 0).astype(jnp.float32)
    g = jnp.dot(ph, wm_s[...], preferred_element_type=jnp.float32)
    rowid = lax.broadcasted_iota(jnp.int32, (BN, NSAMP), 0) + j * BN
    keys = (g + 40.0) * ENC + rowid.astype(jnp.float32)

    runv = run_s[...]
    prev = jnp.full((1, NSAMP), -1.0, jnp.float32)
    for k in range(K):
        bm = jnp.min(jnp.where(keys > prev, keys, BIG), axis=0, keepdims=True)
        rm = jnp.min(jnp.where(runv > prev, runv, BIG), axis=0, keepdims=True)
        m = jnp.minimum(bm, rm)
        run_s[k:k + 1, :] = m
        prev = m

    @pl.when(j == NBLK - 1)
    def _():
        rv = run_s[...]
        idxf = rv - jnp.floor(rv / ENC) * ENC
        base = (b * N).astype(jnp.float32)
        rowsel = lax.broadcasted_iota(jnp.int32, (8, NSAMP), 0) < K
        flat = jnp.where(rowsel, idxf + base, base)
        out[0] = flat.astype(jnp.int32)


def _topk_call(points, lsh_projT, lsh_proj, refc, dvte, scr, offr):
    return pl.pallas_call(
        _topk_body,
        grid=(B, NBLK),
        in_specs=[
            pl.BlockSpec((1, BN, DIM), lambda b, j: (b, j, 0)),
            pl.BlockSpec((DIM, NHASH), lambda b, j: (0, 0)),
            pl.BlockSpec((NHASH, DIM), lambda b, j: (0, 0)),
            pl.BlockSpec((DIM, 1), lambda b, j: (0, 0)),
            pl.BlockSpec((DIM, NSAMP), lambda b, j: (0, 0)),
            pl.BlockSpec((1, NSAMP), lambda b, j: (0, 0)),
            pl.BlockSpec((1, NSAMP), lambda b, j: (0, 0)),
        ],
        out_specs=pl.BlockSpec((1, 8, NSAMP), lambda b, j: (b, 0, 0)),
        out_shape=jax.ShapeDtypeStruct((B, 8, NSAMP), jnp.int32),
        scratch_shapes=[
            pltpu.VMEM((NHASH, NSAMP), jnp.float32),
            pltpu.VMEM((8, NSAMP), jnp.float32),
        ],
    )(points, lsh_projT, lsh_proj, refc, dvte, scr, offr)


_NROWS = B * 8 * NSAMP


def _gather(table, idx_flat):
    info = plsc.get_sparse_core_info()
    nc, ns = info.num_cores, info.num_subcores
    nw = nc * ns
    per_w = _NROWS // nw
    mesh = plsc.VectorSubcoreMesh(core_axis_name="c", subcore_axis_name="s")

    @functools.partial(
        pl.kernel,
        mesh=mesh,
        out_type=jax.ShapeDtypeStruct((_NROWS, DIM), jnp.float32),
        scratch_types=[
            pltpu.VMEM((per_w,), jnp.int32),
            pltpu.VMEM((per_w, DIM), jnp.float32),
            pltpu.SemaphoreType.DMA,
        ],
    )
    def gk(table_hbm, idx_hbm, out_hbm, idx_v, rows_v, sem):
        wid = lax.axis_index("s") * nc + lax.axis_index("c")
        base = wid * per_w
        pltpu.sync_copy(idx_hbm.at[pl.ds(base, per_w)], idx_v)
        pltpu.async_copy(table_hbm.at[idx_v], rows_v, sem).wait()
        pltpu.sync_copy(rows_v, out_hbm.at[pl.ds(base, per_w)])

    return gk(table, idx_flat)


_INV_SQRT_H = 1.0 / math.sqrt(HID)


def _attn_body(nb, refr, dvr, offT, scl, qWT, qb, kWT, kb, vWT, vb, oWT, ob,
               out, attn):
    blocks = []
    for d in range(ND):
        row = dvr[d:d + 1, :] * scl[d:d + 1, 0:1]
        blocks.append(refr[...] + row * offT[:, d:d + 1])
    smp = jnp.concatenate(blocks, axis=0)

    q = jnp.dot(smp, qWT[...], preferred_element_type=jnp.float32) + qb[...]

    slabs = [nb[0, k * NSAMP:(k + 1) * NSAMP, :] for k in range(K)]
    logits = []
    for k in range(K):
        kk = jnp.dot(slabs[k], kWT[...],
                     preferred_element_type=jnp.float32) + kb[...]
        logits.append(jnp.sum(q * kk, axis=1, keepdims=True) * _INV_SQRT_H)

    mx = logits[0]
    for k in range(1, K):
        mx = jnp.maximum(mx, logits[k])
    es = [jnp.exp(l - mx) for l in logits]
    z = es[0]
    for k in range(1, K):
        z = z + es[k]
    ws = [e / z for e in es]

    att = jnp.zeros((NSAMP, HID), jnp.float32)
    for k in range(K):
        vk = jnp.dot(slabs[k], vWT[...],
                     preferred_element_type=jnp.float32) + vb[...]
        att = att + ws[k] * vk

    outr = jnp.dot(att, oWT[...], preferred_element_type=jnp.float32) + ob[...]
    out[...] = jnp.mean(outr, axis=0, keepdims=True)

    lane = lax.broadcasted_iota(jnp.int32, (NSAMP, 8), 1)
    acc = jnp.zeros((NSAMP, 8), jnp.float32)
    for k in range(K):
        acc = jnp.where(lane == k, jnp.broadcast_to(ws[k], (NSAMP, 8)), acc)
    attn[0] = acc


def _attn_call(nbrs, refr, dvr, offT, scl, qWT, qb2, kWT, kb2, vWT, vb2,
               oWT, ob2):
    def full(shape):
        return pl.BlockSpec(shape, lambda *_: tuple(0 for _ in shape))
    return pl.pallas_call(
        _attn_body,
        grid=(B,),
        in_specs=[
            pl.BlockSpec((1, 8 * NSAMP, DIM), lambda b: (b, 0, 0)),
            full((1, DIM)),
            full((ND, DIM)),
            full((NS, ND)),
            full((ND, 1)),
            full((DIM, HID)),
            full((1, HID)),
            full((DIM, HID)),
            full((1, HID)),
            full((DIM, HID)),
            full((1, HID)),
            full((HID, DIM)),
            full((1, DIM)),
        ],
        out_specs=[
            pl.BlockSpec((1, DIM), lambda b: (b, 0)),
            pl.BlockSpec((1, NSAMP, 8), lambda b: (b, 0, 0)),
        ],
        out_shape=[
            jax.ShapeDtypeStruct((B, DIM), jnp.float32),
            jax.ShapeDtypeStruct((B, NSAMP, 8), jnp.float32),
        ],
    )(nbrs, refr, dvr, offT, scl, qWT, qb2, kWT, kb2, vWT, vb2, oWT, ob2)


def kernel(points, ref_point, direction_vectors, sampling_offsets,
           direction_scales, lsh_proj, qW, qb, kW, kb, vW, vb, oW, ob):
    lsh_projT = lsh_proj.T
    refc = ref_point[:, None]
    refr = ref_point[None, :]
    dvte = jnp.repeat(direction_vectors.T, NS, axis=1)
    scr = jnp.repeat(direction_scales, NS)[None, :]
    offr = sampling_offsets.reshape(1, NSAMP)
    offT = sampling_offsets.T
    scl = direction_scales[:, None]

    idx = _topk_call(points, lsh_projT, lsh_proj, refc, dvte, scr, offr)
    nbrs = _gather(points.reshape(B * N, DIM), idx.reshape(_NROWS))
    out, attnp = _attn_call(
        nbrs.reshape(B, 8 * NSAMP, DIM), refr, direction_vectors, offT, scl,
        qW.T, qb[None, :], kW.T, kb[None, :], vW.T, vb[None, :],
        oW.T, ob[None, :])
    return out, attnp[:, :, :K][:, None, :, :]


# speedup vs baseline: 18.7608x; 18.7608x over previous
import functools
import math

import jax
import jax.numpy as jnp
from jax import lax
from jax.experimental import pallas as pl
from jax.experimental.pallas import tpu as pltpu
from jax.experimental.pallas import tpu_sc as plsc

DIM = 128
HID = 256
B = 4
N = 65536
T = 10
FH = 4
NHASH = T * FH
ND = 8
NS = 4
NSAMP = ND * NS
K = 5

BN = 1024
NBLK = N // BN
BIG = 1.0e9
ENC = 131072.0


def _topk_body(pts, projT, projM, refc, dvte, scr, offr, out, wm_s, run_s):
    b = pl.program_id(0)
    j = pl.program_id(1)

    @pl.when(jnp.logical_and(b == 0, j == 0))
    def _():
        sT = refc[...] + (dvte[...] * scr[...]) * offr[...]
        ps = jnp.dot(projM[...], sT, preferred_element_type=jnp.float32)
        wm_s[...] = 1.0 - 2.0 * (ps > 0).astype(jnp.float32)

    @pl.when(j == 0)
    def _():
        run_s[...] = jnp.full((8, NSAMP), BIG, jnp.float32)

    x = pts[0]
    proj = jnp.dot(x, projT[...], preferred_element_type=jnp.float32)
    ph = (proj > 0).astype(jnp.float32)
    g = jnp.dot(ph, wm_s[...], preferred_element_type=jnp.float32)
    rowid = lax.broadcasted_iota(jnp.int32, (BN, NSAMP), 0) + j * BN
    keys = (g + 40.0) * ENC + rowid.astype(jnp.float32)

    runv = run_s[...]
    prev = jnp.full((1, NSAMP), -1.0, jnp.float32)
    for k in range(K):
        bm = jnp.min(jnp.where(keys > prev, keys, BIG), axis=0, keepdims=True)
        rm = jnp.min(jnp.where(runv > prev, runv, BIG), axis=0, keepdims=True)
        m = jnp.minimum(bm, rm)
        run_s[k:k + 1, :] = m
        prev = m

    @pl.when(j == NBLK - 1)
    def _():
        rv = run_s[...]
        idxf = rv - jnp.floor(rv / ENC) * ENC
        base = (b * N).astype(jnp.float32)
        rowsel = lax.broadcasted_iota(jnp.int32, (8, NSAMP), 0) < K
        flat = jnp.where(rowsel, idxf + base, base)
        out[0] = flat.astype(jnp.int32)


def _topk_call(points, lsh_projT, lsh_proj, refc, dvte, scr, offr):
    return pl.pallas_call(
        _topk_body,
        grid=(B, NBLK),
        in_specs=[
            pl.BlockSpec((1, BN, DIM), lambda b, j: (b, j, 0)),
            pl.BlockSpec((DIM, NHASH), lambda b, j: (0, 0)),
            pl.BlockSpec((NHASH, DIM), lambda b, j: (0, 0)),
            pl.BlockSpec((DIM, 1), lambda b, j: (0, 0)),
            pl.BlockSpec((DIM, NSAMP), lambda b, j: (0, 0)),
            pl.BlockSpec((1, NSAMP), lambda b, j: (0, 0)),
            pl.BlockSpec((1, NSAMP), lambda b, j: (0, 0)),
        ],
        out_specs=pl.BlockSpec((1, 8, NSAMP), lambda b, j: (b, 0, 0)),
        out_shape=jax.ShapeDtypeStruct((B, 8, NSAMP), jnp.int32),
        scratch_shapes=[
            pltpu.VMEM((NHASH, NSAMP), jnp.float32),
            pltpu.VMEM((8, NSAMP), jnp.float32),
        ],
    )(points, lsh_projT, lsh_proj, refc, dvte, scr, offr)


_NROWS = B * 8 * NSAMP


def _gather(table, idx_flat):
    info = plsc.get_sparse_core_info()
    nc, ns = info.num_cores, info.num_subcores
    nw = nc * ns
    per_w = _NROWS // nw
    mesh = plsc.VectorSubcoreMesh(core_axis_name="c", subcore_axis_name="s")

    @functools.partial(
        pl.kernel,
        mesh=mesh,
        out_type=jax.ShapeDtypeStruct((_NROWS, DIM), jnp.float32),
        scratch_types=[
            pltpu.VMEM((per_w,), jnp.int32),
            pltpu.VMEM((per_w, DIM), jnp.float32),
            pltpu.SemaphoreType.DMA,
        ],
    )
    def gk(table_hbm, idx_hbm, out_hbm, idx_v, rows_v, sem):
        wid = lax.axis_index("s") * nc + lax.axis_index("c")
        base = wid * per_w
        pltpu.sync_copy(idx_hbm.at[pl.ds(base, per_w)], idx_v)
        pltpu.async_copy(table_hbm.at[idx_v], rows_v, sem).wait()
        pltpu.sync_copy(rows_v, out_hbm.at[pl.ds(base, per_w)])

    return gk(table, idx_flat)


_INV_SQRT_H = 1.0 / math.sqrt(HID)


def _attn_body(nb, refr, dvr, offT, scl, qWT, qb, kWT, kb, vWT, vb, oWT, ob,
               out, attn):
    blocks = []
    for d in range(ND):
        row = dvr[d:d + 1, :] * scl[d:d + 1, 0:1]
        blocks.append(refr[...] + row * offT[:, d:d + 1])
    smp = jnp.concatenate(blocks, axis=0)

    q = jnp.dot(smp, qWT[...], preferred_element_type=jnp.float32) + qb[...]

    slabs = [nb[0, k * NSAMP:(k + 1) * NSAMP, :] for k in range(K)]
    logits = []
    for k in range(K):
        kk = jnp.dot(slabs[k], kWT[...],
                     preferred_element_type=jnp.float32) + kb[...]
        logits.append(jnp.sum(q * kk, axis=1, keepdims=True) * _INV_SQRT_H)

    mx = logits[0]
    for k in range(1, K):
        mx = jnp.maximum(mx, logits[k])
    es = [jnp.exp(l - mx) for l in logits]
    z = es[0]
    for k in range(1, K):
        z = z + es[k]
    ws = [e / z for e in es]

    att = jnp.zeros((NSAMP, HID), jnp.float32)
    for k in range(K):
        vk = jnp.dot(slabs[k], vWT[...],
                     preferred_element_type=jnp.float32) + vb[...]
        att = att + ws[k] * vk

    outr = jnp.dot(att, oWT[...], preferred_element_type=jnp.float32) + ob[...]
    mean = jnp.mean(outr, axis=0, keepdims=True)
    out[0] = jnp.broadcast_to(mean, (8, DIM))

    lane = lax.broadcasted_iota(jnp.int32, (NSAMP, 8), 1)
    acc = jnp.zeros((NSAMP, 8), jnp.float32)
    for k in range(K):
        acc = jnp.where(lane == k, jnp.broadcast_to(ws[k], (NSAMP, 8)), acc)
    attn[0] = acc


def _attn_call(nbrs, refr, dvr, offT, scl, qWT, qb2, kWT, kb2, vWT, vb2,
               oWT, ob2):
    def full(shape):
        return pl.BlockSpec(shape, lambda *_: tuple(0 for _ in shape))
    return pl.pallas_call(
        _attn_body,
        grid=(B,),
        in_specs=[
            pl.BlockSpec((1, 8 * NSAMP, DIM), lambda b: (b, 0, 0)),
            full((1, DIM)),
            full((ND, DIM)),
            full((NS, ND)),
            full((ND, 1)),
            full((DIM, HID)),
            full((1, HID)),
            full((DIM, HID)),
            full((1, HID)),
            full((DIM, HID)),
            full((1, HID)),
            full((HID, DIM)),
            full((1, DIM)),
        ],
        out_specs=[
            pl.BlockSpec((1, 8, DIM), lambda b: (b, 0, 0)),
            pl.BlockSpec((1, NSAMP, 8), lambda b: (b, 0, 0)),
        ],
        out_shape=[
            jax.ShapeDtypeStruct((B, 8, DIM), jnp.float32),
            jax.ShapeDtypeStruct((B, NSAMP, 8), jnp.float32),
        ],
    )(nbrs, refr, dvr, offT, scl, qWT, qb2, kWT, kb2, vWT, vb2, oWT, ob2)


def kernel(points, ref_point, direction_vectors, sampling_offsets,
           direction_scales, lsh_proj, qW, qb, kW, kb, vW, vb, oW, ob):
    lsh_projT = lsh_proj.T
    refc = ref_point[:, None]
    refr = ref_point[None, :]
    dvte = jnp.repeat(direction_vectors.T, NS, axis=1)
    scr = jnp.repeat(direction_scales, NS)[None, :]
    offr = sampling_offsets.reshape(1, NSAMP)
    offT = sampling_offsets.T
    scl = direction_scales[:, None]

    idx = _topk_call(points, lsh_projT, lsh_proj, refc, dvte, scr, offr)
    nbrs = _gather(points.reshape(B * N, DIM), idx.reshape(_NROWS))
    out, attnp = _attn_call(
        nbrs.reshape(B, 8 * NSAMP, DIM), refr, direction_vectors, offT, scl,
        qW.T, qb[None, :], kW.T, kb[None, :], vW.T, vb[None, :],
        oW.T, ob[None, :])
    return out[:, 0, :], attnp[:, :, :K][:, None, :, :]
